# baseline (speedup 1.0000x reference)
import functools

import jax
import jax.numpy as jnp
from jax import lax
from jax.experimental import pallas as pl
from jax.experimental.pallas import tpu as pltpu
from jax.experimental.pallas import tpu_sc as plsc

_LANES = 16


@functools.lru_cache(maxsize=None)
def _make_gather(num_entities, batch, dim, nc, ns):
    nw = nc * ns
    b_per_w = batch // nw
    n_grp = b_per_w // _LANES
    mesh = plsc.VectorSubcoreMesh(core_axis_name="c", subcore_axis_name="s")

    @functools.partial(
        pl.kernel,
        out_type=jax.ShapeDtypeStruct((batch * dim,), jnp.float32),
        mesh=mesh,
        scratch_types=(
            [pltpu.VMEM((b_per_w,), jnp.int32)]
            + [pltpu.VMEM((dim, 128), jnp.float32) for _ in range(8)]
            + [pltpu.VMEM((b_per_w * dim,), jnp.float32)]
            + [pltpu.SemaphoreType.DMA for _ in range(8)]
        ),
        compiler_params=pltpu.CompilerParams(
            disable_bounds_checks=True, needs_layout_passes=False
        ),
    )
    def gather_kernel(idx_hbm, tab_hbm, out_hbm, *refs):
        idx_v = refs[0]
        bufs = refs[1:9]
        rows_v = refs[9]
        sems = refs[10:18]
        ndeep = 8
        wid = lax.axis_index("s") * nc + lax.axis_index("c")
        base = wid * b_per_w
        pltpu.sync_copy(idx_hbm.at[pl.ds(base, b_per_w)], idx_v)
        iota = lax.iota(jnp.int32, _LANES)

        def start(v, par):
            off = pl.multiple_of((v >> 7) * 128, 128)
            pltpu.async_copy(tab_hbm.at[:, pl.ds(off, 128)], bufs[par], sems[par])

        def finish(j, v, par):
            pltpu.make_async_copy(
                tab_hbm.at[:, pl.ds(0, 128)], bufs[par], sems[par]
            ).wait()
            lane = jnp.full((_LANES,), v & 127, jnp.int32)
            for k in range(dim // _LANES):
                vals = plsc.load_gather(bufs[par], [iota + (k * _LANES), lane])
                rows_v[pl.ds(j * dim + k * _LANES, _LANES)] = vals

        vec0 = idx_v[pl.ds(0, _LANES)]
        for l in range(ndeep):
            start(vec0[l], l)

        @pl.loop(0, n_grp, init_carry=vec0)
        def _grp(g, vec):
            nxt_off = jnp.minimum((g + 1) * _LANES, b_per_w - _LANES)
            vec_n = idx_v[pl.ds(nxt_off, _LANES)]
            for l in range(_LANES):
                j = g * _LANES + l
                finish(j, vec[l], l % ndeep)
                if l < _LANES - ndeep:
                    start(vec[l + ndeep], (l + ndeep) % ndeep)
                else:

                    @pl.when(g < n_grp - 1)
                    def _():
                        start(vec_n[l + ndeep - _LANES], (l + ndeep) % ndeep)

            return vec_n

        pltpu.sync_copy(rows_v, out_hbm.at[pl.ds(base * dim, b_per_w * dim)])

    return gather_kernel


def kernel(elements, entity_embeddings):
    (batch,) = elements.shape
    num_entities, dim = entity_embeddings.shape
    info = plsc.get_sparse_core_info()
    fn = _make_gather(num_entities, batch, dim, info.num_cores, info.num_subcores)
    flat = fn(elements, entity_embeddings.T)
    return flat.reshape(batch, dim)


# speedup vs baseline: 1.0492x; 1.0492x over previous
import functools

import jax
import jax.numpy as jnp
from jax import lax
from jax.experimental import pallas as pl
from jax.experimental.pallas import tpu as pltpu
from jax.experimental.pallas import tpu_sc as plsc

_LANES = 16
_NDEEP = 8
_NQ = 4


@functools.lru_cache(maxsize=None)
def _make_gather(num_entities, batch, dim, nc, ns):
    nw = nc * ns
    b_per_w = batch // nw
    n_grp = b_per_w // _LANES
    grp_per_q = n_grp // _NQ
    mesh = plsc.VectorSubcoreMesh(core_axis_name="c", subcore_axis_name="s")

    @functools.partial(
        pl.kernel,
        out_type=jax.ShapeDtypeStruct((dim, batch), jnp.float32),
        mesh=mesh,
        scratch_types=(
            [pltpu.VMEM((b_per_w,), jnp.int32)]
            + [pltpu.VMEM((dim, 128), jnp.float32) for _ in range(_NDEEP)]
            + [pltpu.VMEM((dim, 128), jnp.float32) for _ in range(_NQ)]
            + [pltpu.SemaphoreType.DMA for _ in range(_NDEEP)]
            + [pltpu.SemaphoreType.DMA]
        ),
        compiler_params=pltpu.CompilerParams(
            disable_bounds_checks=True, needs_layout_passes=False
        ),
    )
    def gather_kernel(idx_hbm, tab_hbm, out_hbm, *refs):
        idx_v = refs[0]
        bufs = refs[1 : 1 + _NDEEP]
        qbufs = refs[1 + _NDEEP : 1 + _NDEEP + _NQ]
        sems = refs[1 + _NDEEP + _NQ : 1 + 2 * _NDEEP + _NQ]
        osem = refs[1 + 2 * _NDEEP + _NQ]
        wid = lax.axis_index("s") * nc + lax.axis_index("c")
        base = wid * b_per_w
        pltpu.sync_copy(idx_hbm.at[pl.ds(base, b_per_w)], idx_v)
        iota = lax.iota(jnp.int32, _LANES)

        def start(v, par):
            off = pl.multiple_of((v >> 7) * 128, 128)
            pltpu.async_copy(tab_hbm.at[:, pl.ds(off, 128)], bufs[par], sems[par])

        def finish(gl, l, v, par, qbuf):
            pltpu.make_async_copy(
                tab_hbm.at[:, pl.ds(0, 128)], bufs[par], sems[par]
            ).wait()
            lane = jnp.full((_LANES,), v & 127, jnp.int32)
            col = jnp.full((_LANES,), gl * _LANES + l, jnp.int32)
            for k in range(dim // _LANES):
                vals = plsc.load_gather(bufs[par], [iota + (k * _LANES), lane])
                plsc.store_scatter(qbuf, [iota + (k * _LANES), col], vals)

        vec0 = idx_v[pl.ds(0, _LANES)]
        for l in range(_NDEEP):
            start(vec0[l], l)

        vec = vec0
        for q in range(_NQ):

            @pl.loop(0, grp_per_q, init_carry=vec)
            def _grp(g, vec, q=q):
                gg = q * grp_per_q + g
                nxt_off = jnp.minimum((gg + 1) * _LANES, b_per_w - _LANES)
                vec_n = idx_v[pl.ds(nxt_off, _LANES)]
                for l in range(_LANES):
                    finish(g, l, vec[l], l % _NDEEP, qbufs[q])
                    if l < _LANES - _NDEEP:
                        start(vec[l + _NDEEP], (l + _NDEEP) % _NDEEP)
                    else:

                        @pl.when(gg < n_grp - 1)
                        def _():
                            start(vec_n[l + _NDEEP - _LANES], (l + _NDEEP) % _NDEEP)

                return vec_n

            vec = _grp
            pltpu.async_copy(
                qbufs[q],
                out_hbm.at[:, pl.ds(base + q * 128, 128)],
                osem,
            )

        for q in range(_NQ):
            pltpu.make_async_copy(
                qbufs[q], out_hbm.at[:, pl.ds(base + q * 128, 128)], osem
            ).wait()

    return gather_kernel


def kernel(elements, entity_embeddings):
    (batch,) = elements.shape
    num_entities, dim = entity_embeddings.shape
    info = plsc.get_sparse_core_info()
    fn = _make_gather(num_entities, batch, dim, info.num_cores, info.num_subcores)
    out_t = fn(elements, entity_embeddings.T)
    return out_t.T
